# baseline (speedup 1.0000x reference)
import functools

import jax
import jax.numpy as jnp
from jax import lax
from jax.experimental import pallas as pl
from jax.experimental.pallas import tpu as pltpu
from jax.experimental.pallas import tpu_sc as plsc

D_EMB = 64
RPS = 128
K = 5
CHUNK = K * RPS


@functools.partial(jax.jit, static_argnums=(2, 3))
def _embedding_gather(idx2d, weight, n_total, n_workers):
    per_w = n_total // n_workers
    idx_rows = per_w // RPS
    n_chunks = per_w // CHUNK

    mesh = plsc.VectorSubcoreMesh(core_axis_name="c", subcore_axis_name="s")

    @functools.partial(
        pl.kernel,
        out_type=jax.ShapeDtypeStruct((n_total, D_EMB), jnp.float32),
        mesh=mesh,
        compiler_params=pltpu.CompilerParams(use_tc_tiling_on_sc=False),
        scratch_types=[
            pltpu.VMEM((idx_rows, RPS), jnp.int32),
            pltpu.VMEM((CHUNK, D_EMB), jnp.float32),
            pltpu.SemaphoreType.DMA,
        ],
    )
    def k(idx_hbm, table_hbm, out_hbm, idx_v, rows_v, sem):
        nc = 2
        wid = lax.axis_index("s") * nc + lax.axis_index("c")
        base = wid * per_w

        pltpu.sync_copy(idx_hbm.at[pl.ds(wid * idx_rows, idx_rows)], idx_v)

        def body(g, carry):
            descs = []
            for j in range(K):
                d = pltpu.async_copy(
                    table_hbm.at[idx_v.at[g * K + j]],
                    rows_v.at[pl.ds(j * RPS, RPS)],
                    sem,
                )
                descs.append(d)
            for d in descs:
                d.wait()
            pltpu.sync_copy(rows_v, out_hbm.at[pl.ds(base + g * CHUNK, CHUNK)])
            return carry

        lax.fori_loop(0, n_chunks, body, 0)

    return k(idx2d, weight)


def kernel(indices, weight):
    b, h = indices.shape
    n_total = b * h
    idx2d = indices.reshape(n_total // RPS, RPS)
    out = _embedding_gather(idx2d, weight, n_total, 32)
    return out.reshape(b, h, D_EMB)


# speedup vs baseline: 1.0128x; 1.0128x over previous
import functools

import jax
import jax.numpy as jnp
from jax import lax
from jax.experimental import pallas as pl
from jax.experimental.pallas import tpu as pltpu
from jax.experimental.pallas import tpu_sc as plsc

D_EMB = 64
RPS = 128
K = 5
CHUNK = K * RPS


@functools.partial(jax.jit, static_argnums=(2, 3))
def _embedding_gather(idx2d, weight, n_total, n_workers):
    per_w = n_total // n_workers
    idx_rows = per_w // RPS
    n_chunks = per_w // CHUNK
    n_pairs = n_chunks // 2

    mesh = plsc.VectorSubcoreMesh(core_axis_name="c", subcore_axis_name="s")

    @functools.partial(
        pl.kernel,
        out_type=jax.ShapeDtypeStruct((n_total, D_EMB), jnp.float32),
        mesh=mesh,
        compiler_params=pltpu.CompilerParams(use_tc_tiling_on_sc=False),
        scratch_types=[
            pltpu.VMEM((idx_rows, RPS), jnp.int32),
            pltpu.VMEM((2 * CHUNK, D_EMB), jnp.float32),
            pltpu.SemaphoreType.DMA,
            pltpu.SemaphoreType.DMA,
            pltpu.SemaphoreType.DMA,
            pltpu.SemaphoreType.DMA,
        ],
    )
    def k(idx_hbm, table_hbm, out_hbm, idx_v, rows_v, gsem0, gsem1, ssem0, ssem1):
        nc = 2
        wid = lax.axis_index("s") * nc + lax.axis_index("c")
        base = wid * per_w

        pltpu.sync_copy(idx_hbm.at[pl.ds(wid * idx_rows, idx_rows)], idx_v)

        def fire_gather(g, slot, sem):
            descs = []
            for j in range(K):
                descs.append(pltpu.async_copy(
                    table_hbm.at[idx_v.at[g * K + j]],
                    rows_v.at[pl.ds(slot * CHUNK + j * RPS, RPS)],
                    sem,
                ))
            return descs

        def buf(slot):
            return rows_v.at[pl.ds(slot * CHUNK, CHUNK)]

        def drain(sem):
            pltpu.make_async_copy(table_hbm.at[pl.ds(0, CHUNK)], buf(0), sem).wait()

        def fire_store(g, slot, sem):
            return pltpu.async_copy(buf(slot), out_hbm.at[pl.ds(base + g * CHUNK, CHUNK)], sem)

        fire_gather(0, 0, gsem0)

        def body(p, carry):
            a = 2 * p
            drain(gsem0)
            st_a = fire_store(a, 0, ssem0)

            @pl.when(p > 0)
            def _():
                drain(ssem1)

            gb = fire_gather(a + 1, 1, gsem1)
            st_a.wait()

            @pl.when(p < n_pairs - 1)
            def _():
                fire_gather(a + 2, 0, gsem0)

            for d in gb:
                d.wait()
            fire_store(a + 1, 1, ssem1)
            return carry

        lax.fori_loop(0, n_pairs, body, 0)
        drain(ssem1)

    return k(idx2d, weight)


def kernel(indices, weight):
    b, h = indices.shape
    n_total = b * h
    idx2d = indices.reshape(n_total // RPS, RPS)
    out = _embedding_gather(idx2d, weight, n_total, 32)
    return out.reshape(b, h, D_EMB)


# speedup vs baseline: 1.0140x; 1.0012x over previous
import functools

import jax
import jax.numpy as jnp
from jax import lax
from jax.experimental import pallas as pl
from jax.experimental.pallas import tpu as pltpu
from jax.experimental.pallas import tpu_sc as plsc

D_EMB = 64
BPC = 4
N_WORKERS = 32


@functools.partial(jax.jit, static_argnums=(2,))
def _embedding_gather(indices, weight, hist):
    batch = indices.shape[0]
    b_per_w = batch // N_WORKERS
    n_chunks = b_per_w // BPC
    n_pairs = n_chunks // 2
    h0 = min(128, hist)
    splits = [(0, h0)] + ([(h0, hist - h0)] if hist > h0 else [])

    mesh = plsc.VectorSubcoreMesh(core_axis_name="c", subcore_axis_name="s")

    @functools.partial(
        pl.kernel,
        out_type=jax.ShapeDtypeStruct((batch, hist, D_EMB), jnp.float32),
        mesh=mesh,
        compiler_params=pltpu.CompilerParams(use_tc_tiling_on_sc=False),
        scratch_types=[
            pltpu.VMEM((b_per_w, hist), jnp.int32),
            pltpu.VMEM((2 * BPC, hist, D_EMB), jnp.float32),
            pltpu.SemaphoreType.DMA,
            pltpu.SemaphoreType.DMA,
            pltpu.SemaphoreType.DMA,
            pltpu.SemaphoreType.DMA,
        ],
    )
    def k(idx_hbm, table_hbm, out_hbm, idx_v, rows_v, gsem0, gsem1, ssem0, ssem1):
        nc = 2
        wid = lax.axis_index("s") * nc + lax.axis_index("c")
        b0 = wid * b_per_w

        pltpu.sync_copy(idx_hbm.at[pl.ds(b0, b_per_w)], idx_v)

        def fire_gather(g, slot, sem):
            descs = []
            for r in range(BPC):
                for off, ln in splits:
                    descs.append(pltpu.async_copy(
                        table_hbm.at[idx_v.at[g * BPC + r, pl.ds(off, ln)]],
                        rows_v.at[slot * BPC + r, pl.ds(off, ln), :],
                        sem,
                    ))
            return descs

        def buf(slot):
            return rows_v.at[pl.ds(slot * BPC, BPC)]

        def drain(sem):
            pltpu.make_async_copy(out_hbm.at[pl.ds(0, BPC)], buf(0), sem).wait()

        def fire_store(g, slot, sem):
            return pltpu.async_copy(
                buf(slot), out_hbm.at[pl.ds(b0 + g * BPC, BPC)], sem)

        fire_gather(0, 0, gsem0)

        def body(p, carry):
            a = 2 * p
            drain(gsem0)
            st_a = fire_store(a, 0, ssem0)

            @pl.when(p > 0)
            def _():
                drain(ssem1)

            gb = fire_gather(a + 1, 1, gsem1)
            st_a.wait()

            @pl.when(p < n_pairs - 1)
            def _():
                fire_gather(a + 2, 0, gsem0)

            for d in gb:
                d.wait()
            fire_store(a + 1, 1, ssem1)
            return carry

        lax.fori_loop(0, n_pairs, body, 0)
        drain(ssem1)

    return k(indices, weight)


def kernel(indices, weight):
    return _embedding_gather(indices, weight, indices.shape[1])
